# baseline (speedup 1.0000x reference)
import functools

import jax
import jax.numpy as jnp
from jax import lax
from jax.experimental import pallas as pl
from jax.experimental.pallas import tpu as pltpu
from jax.experimental.pallas import tpu_sc as plsc

_B = 16384
_DIM = 64
_GAMMA = 12.0
_NC = 2
_NS = 16
_NW = _NC * _NS
_BW = _B // _NW
_NCHUNK = 4
_CH = _BW // _NCHUNK
_PASS = 256
_CPP = _PASS // _CH
_NPASS = _BW // _PASS
_RPB = 16
_NGP = _PASS // _RPB


def _lane_shuffle(x, idx):
    dnums = lax.GatherDimensionNumbers(
        offset_dims=(), collapsed_slice_dims=(0,), start_index_map=(0,))
    return lax.gather(x, idx[:, None], dnums, (1,),
                      mode=lax.GatherScatterMode.PROMISE_IN_BOUNDS)


def _score_body(heads_hbm, rel_hbm, tails_hbm, ent2_hbm, rel2_hbm, out_hbm,
                hraw, rraw, traw, h2, t2, r2, outv, sem):
    wid = lax.axis_index("s") * _NC + lax.axis_index("c")
    base = wid * _BW
    lane = lax.iota(jnp.int32, 16)

    for c in range(_NCHUNK):
        off = base + c * _CH
        pltpu.sync_copy(heads_hbm.at[pl.ds(off, _CH)], hraw.at[c])
        pltpu.sync_copy(rel_hbm.at[pl.ds(off, _CH)], rraw.at[c])
        pltpu.sync_copy(tails_hbm.at[pl.ds(off, _CH)], traw.at[c])

    for p in range(_NPASS):
        copies = []
        for c2 in range(_CPP):
            c = p * _CPP + c2
            dst = pl.ds(c2 * _CH, _CH)
            copies.append(pltpu.async_copy(ent2_hbm.at[hraw.at[c]], h2.at[dst], sem))
            copies.append(pltpu.async_copy(ent2_hbm.at[traw.at[c]], t2.at[dst], sem))
            copies.append(pltpu.async_copy(rel2_hbm.at[rraw.at[c]], r2.at[dst], sem))
        for cp in copies:
            cp.wait()

        def grp(g, carry):
            out16 = jnp.zeros((16,), jnp.float32)
            for ri in range(_RPB):
                row = g * _RPB + ri
                s = jnp.zeros((16,), jnp.float32)
                for q in range(_DIM // 16):
                    sl = pl.ds(q * 16, 16)
                    s = s + jnp.abs(h2[row, sl] + r2[row, sl] - t2[row, sl])
                for sh in (8, 4, 2, 1):
                    s = s + _lane_shuffle(s, lane ^ sh)
                out16 = jnp.where(lane == ri, s, out16)
            outv[pl.ds(p * _PASS + g * _RPB, _RPB)] = _GAMMA - out16
            return carry

        lax.fori_loop(0, _NGP, grp, 0)

    pltpu.sync_copy(outv, out_hbm.at[pl.ds(base, _BW)])


@functools.partial(
    pl.kernel,
    out_type=jax.ShapeDtypeStruct((_B,), jnp.float32),
    mesh=plsc.VectorSubcoreMesh(core_axis_name="c", subcore_axis_name="s"),
    compiler_params=pltpu.CompilerParams(use_tc_tiling_on_sc=True),
    scratch_types=[
        pltpu.VMEM((_NCHUNK, _CH), jnp.int32),
        pltpu.VMEM((_NCHUNK, _CH), jnp.int32),
        pltpu.VMEM((_NCHUNK, _CH), jnp.int32),
        pltpu.VMEM((_PASS, 2 * _DIM), jnp.float32),
        pltpu.VMEM((_PASS, 2 * _DIM), jnp.float32),
        pltpu.VMEM((_PASS, 2 * _DIM), jnp.float32),
        pltpu.VMEM((_BW,), jnp.float32),
        pltpu.SemaphoreType.DMA,
    ],
)
def _score(*refs):
    _score_body(*refs)


def kernel(heads, relations, tails, entity_embedding, relation_embedding):
    ent2 = jnp.concatenate(
        [entity_embedding, jnp.zeros_like(entity_embedding)], axis=1)
    rel2 = jnp.pad(relation_embedding, ((0, 0), (0, _DIM)))
    return _score(heads.astype(jnp.int32), relations.astype(jnp.int32),
                  tails.astype(jnp.int32), ent2, rel2)
